# baseline (speedup 1.0000x reference)
import functools

import jax
import jax.numpy as jnp
from jax import lax
from jax.experimental import pallas as pl
from jax.experimental.pallas import tpu as pltpu
from jax.experimental.pallas import tpu_sc as plsc

N = 10000
NPAD = 10240
D = 128
B = 64
C = 10

NC = 2
NS = 16
NW = NC * NS

K = 32
EPT = 10240
NBATCH = EPT // K
E2 = NW * EPT
EROW = 2 * K
TWORDS = NBATCH * EROW
TPAD = 20736
ROWS_PT = NPAD // NS

_f32 = jnp.float32
_i32 = jnp.int32

_MESH = plsc.VectorSubcoreMesh(
    core_axis_name="c", subcore_axis_name="s", num_cores=NC, num_subcores=NS)


@functools.partial(
    pl.kernel,
    out_type=jax.ShapeDtypeStruct((NW, NPAD), _f32),
    mesh=_MESH,
    compiler_params=pltpu.CompilerParams(needs_layout_passes=False),
    scratch_types=[
        pltpu.VMEM((TPAD,), _i32),
        pltpu.VMEM((NPAD,), _f32),
    ],
)
def _sc_deg(edata_hbm, out_hbm, ed_t, deg_t):
    c = lax.axis_index("c")
    s = lax.axis_index("s")
    w = c * NS + s
    pltpu.sync_copy(edata_hbm.at[w], ed_t)

    zero16 = jnp.zeros((16,), _f32)

    def _zero(i, _):
        deg_t[pl.ds(pl.multiple_of(i * 16, 16), 16)] = zero16
        return _

    lax.fori_loop(0, NPAD // 16, _zero, None)

    def _acc(m, _):
        off = pl.multiple_of(m * 4 * K, 128)
        for q in range(2 * K // 16):
            col_v = ed_t[pl.ds(off + q * 16, 16)] & 0xFFFF
            ew_v = plsc.bitcast(ed_t[pl.ds(off + 2 * K + q * 16, 16)], _f32)
            plsc.addupdate_scatter(deg_t, [col_v], ew_v)
        return _

    lax.fori_loop(0, NBATCH // 2, _acc, None)
    pltpu.sync_copy(deg_t, out_hbm.at[w])


@functools.partial(
    pl.kernel,
    out_type=jax.ShapeDtypeStruct((NC, NPAD, D), _f32),
    mesh=_MESH,
    compiler_params=pltpu.CompilerParams(needs_layout_passes=False),
    scratch_types=[
        pltpu.VMEM((TPAD,), _i32),
        pltpu.VMEM((2 * K, D), _f32),
        pltpu.VMEM((2 * K, D), _f32),
        pltpu.VMEM((K, D), _f32),
        pltpu.VMEM((K, D), _f32),
        pltpu.VMEM((2 * K,), _i32),
        pltpu.VMEM((2 * K,), _i32),
        pltpu.VMEM((K,), _i32),
        pltpu.VMEM((K,), _i32),
        pltpu.VMEM_SHARED((NPAD, D), _f32),
        pltpu.SemaphoreType.DMA,
        pltpu.SemaphoreType.DMA,
        pltpu.SemaphoreType.DMA,
        pltpu.SemaphoreType.DMA,
        pltpu.SemaphoreType.DMA,
    ],
)
def _sc_prop(g_hbm, edata_hbm, out_hbm,
             ed_t, gb0, gb1, sb0, sb1,
             rb0, rb1, cb0, cb1, acc,
             gs0, gs1, ss0, ss1, es):
    c = lax.axis_index("c")
    s = lax.axis_index("s")
    w = c * NS + s
    npair = NBATCH // 2
    prow = 2 * EROW

    gbufs = (gb0, gb1)
    rbufs = (rb0, rb1)
    gsems = (gs0, gs1)
    sbufs = (sb0, sb1)
    cbufs = (cb0, cb1)
    ssems = (ss0, ss1)

    pltpu.async_copy(edata_hbm.at[w], ed_t, es)

    zero16 = jnp.zeros((16,), _f32)
    for j in range(K):
        for t in range(D // 16):
            sl = pl.ds(t * 16, 16)
            sb0[j, sl] = zero16
            sb1[j, sl] = zero16
    zbase = s * ROWS_PT
    for t in range(ROWS_PT // K):
        pltpu.sync_copy(sbufs[t % 2], acc.at[pl.ds(zbase + t * K, K)])
    plsc.subcore_barrier()

    pltpu.make_async_copy(edata_hbm.at[w], ed_t, es).wait()

    def _start_gather(u, pair):
        off = pl.multiple_of(pair * prow, 128)
        for q in range(2 * K // 16):
            rbufs[u][pl.ds(q * 16, 16)] = (
                ed_t[pl.ds(off + q * 16, 16)] >> 16)
        pltpu.async_copy(g_hbm.at[rbufs[u]], gbufs[u], gsems[u])

    _start_gather(0, 0)
    _start_gather(1, 1)

    def _body(i, _):
        for u in range(2):
            pair = i * 2 + u
            off = pl.multiple_of(pair * prow, 128)

            pltpu.make_async_copy(g_hbm.at[rbufs[u]], gbufs[u], gsems[u]).wait()

            for h in range(2):
                def _wait_scatter(h=h):
                    pltpu.make_async_copy(
                        sbufs[h], acc.at[cbufs[h]], ssems[h]).wait()
                if u == 1:
                    _wait_scatter()
                else:
                    pl.when(i >= 1)(_wait_scatter)

                hoff = off + h * K
                for j in range(K):
                    if j % 16 == 0:
                        ew_v = plsc.bitcast(
                            ed_t[pl.ds(off + 2 * K + h * K + j, 16)], _f32)
                    wj = jnp.full((16,), ew_v[j % 16], _f32)
                    for t in range(D // 16):
                        sl = pl.ds(t * 16, 16)
                        sbufs[h][j, sl] = gbufs[u][h * K + j, sl] * wj

                for q in range(K // 16):
                    cbufs[h][pl.ds(q * 16, 16)] = (
                        ed_t[pl.ds(hoff + q * 16, 16)] & 0xFFFF)

                pltpu.async_copy(sbufs[h], acc.at[cbufs[h]], ssems[h],
                                 add=True)

            _start_gather(u, pair + 2)
        return _

    lax.fori_loop(0, npair // 2, _body, None)

    for h in range(2):
        pltpu.make_async_copy(sbufs[h], acc.at[cbufs[h]], ssems[h]).wait()
    for u in range(2):
        pltpu.make_async_copy(g_hbm.at[rbufs[u]], gbufs[u], gsems[u]).wait()

    plsc.subcore_barrier()
    dbase = s * ROWS_PT
    pltpu.sync_copy(acc.at[pl.ds(dbase, ROWS_PT)],
                    out_hbm.at[c, pl.ds(dbase, ROWS_PT)])


def _tc_dinv_body(parts_ref, out_ref):
    deg = jnp.sum(parts_ref[...], axis=0) + 1.0
    out_ref[...] = lax.rsqrt(deg)


def _tc_dinv(parts):
    return pl.pallas_call(
        _tc_dinv_body,
        out_shape=jax.ShapeDtypeStruct((NPAD,), _f32),
    )(parts)


_BM = 1024
_GRID = NPAD // _BM


def _tc_mm_body(x_ref, w_ref, dinv_ref, g_ref):
    g_ref[...] = dinv_ref[...] * jnp.dot(
        x_ref[...], w_ref[...], preferred_element_type=_f32)


def _tc_mm(x, w, dinv2):
    return pl.pallas_call(
        _tc_mm_body,
        grid=(_GRID,),
        in_specs=[
            pl.BlockSpec((_BM, D), lambda i: (i, 0)),
            pl.BlockSpec((D, D), lambda i: (0, 0)),
            pl.BlockSpec((_BM, 1), lambda i: (i, 0)),
        ],
        out_specs=pl.BlockSpec((_BM, D), lambda i: (i, 0)),
        out_shape=jax.ShapeDtypeStruct((NPAD, D), _f32),
    )(x, w, dinv2)


def _tc_combine_mm_body(with_res, p_ref, g_ref, dinv_ref, b_ref, w_ref,
                        res_ref, t_ref, gn_ref):
    ps = p_ref[0] + p_ref[1]
    t = dinv_ref[...] * (ps + g_ref[...]) + b_ref[...]
    if with_res:
        t = t + res_ref[...]
    t = jnp.maximum(t, 0.0)
    t_ref[...] = t
    gn_ref[...] = dinv_ref[...] * jnp.dot(
        t, w_ref[...], preferred_element_type=_f32)


def _tc_combine_mm(p, g, dinv2, b2d, w, res):
    with_res = res is not None
    specs = [
        pl.BlockSpec((NC, _BM, D), lambda i: (0, i, 0)),
        pl.BlockSpec((_BM, D), lambda i: (i, 0)),
        pl.BlockSpec((_BM, 1), lambda i: (i, 0)),
        pl.BlockSpec((1, D), lambda i: (0, 0)),
        pl.BlockSpec((D, D), lambda i: (0, 0)),
    ]
    args = [p, g, dinv2, b2d, w]
    if with_res:
        specs.append(pl.BlockSpec((_BM, D), lambda i: (i, 0)))
        args.append(res)
        body = lambda p_, g_, d_, b_, w_, r_, t_, gn_: _tc_combine_mm_body(
            True, p_, g_, d_, b_, w_, r_, t_, gn_)
    else:
        body = lambda p_, g_, d_, b_, w_, t_, gn_: _tc_combine_mm_body(
            False, p_, g_, d_, b_, w_, None, t_, gn_)
    return pl.pallas_call(
        body,
        grid=(_GRID,),
        in_specs=specs,
        out_specs=[
            pl.BlockSpec((_BM, D), lambda i: (i, 0)),
            pl.BlockSpec((_BM, D), lambda i: (i, 0)),
        ],
        out_shape=[
            jax.ShapeDtypeStruct((NPAD, D), _f32),
            jax.ShapeDtypeStruct((NPAD, D), _f32),
        ],
    )(*args)


def _tc_combine_body(p_ref, g_ref, dinv_ref, b_ref, res_ref, h_ref):
    ps = p_ref[0] + p_ref[1]
    t = res_ref[...] + dinv_ref[...] * (ps + g_ref[...]) + b_ref[...]
    h_ref[...] = jnp.maximum(t, 0.0)


def _tc_combine(p, g, dinv2, b2d, res):
    return pl.pallas_call(
        _tc_combine_body,
        grid=(_GRID,),
        in_specs=[
            pl.BlockSpec((NC, _BM, D), lambda i: (0, i, 0)),
            pl.BlockSpec((_BM, D), lambda i: (i, 0)),
            pl.BlockSpec((_BM, 1), lambda i: (i, 0)),
            pl.BlockSpec((1, D), lambda i: (0, 0)),
            pl.BlockSpec((_BM, D), lambda i: (i, 0)),
        ],
        out_specs=pl.BlockSpec((_BM, D), lambda i: (i, 0)),
        out_shape=jax.ShapeDtypeStruct((NPAD, D), _f32),
    )(p, g, dinv2, b2d, res)


def _tc_readout_body(p_ref, g_ref, dinv_ref, b_ref, res_ref,
                     batch_ref, wo_ref, bo_ref, out_ref):
    t = (res_ref[...] + dinv_ref[...] * (p_ref[0] + p_ref[1] + g_ref[...])
         + b_ref[...])
    h = jnp.maximum(t, 0.0)
    bat = batch_ref[...]
    karr = lax.broadcasted_iota(_i32, (B, 1), 0)
    cnt = jnp.sum((bat <= karr).astype(_i32), axis=1, keepdims=True)
    pos = lax.broadcasted_iota(_i32, (B, NPAD), 1)
    sel = (pos == (cnt - 1)).astype(_f32)
    virt = jnp.dot(sel, h, preferred_element_type=_f32)
    out_ref[...] = jnp.dot(
        virt, wo_ref[...], preferred_element_type=_f32) + bo_ref[...]


def _tc_readout(p, g, dinv2, b2d, res, batch2d, wo_pad, bo2d):
    return pl.pallas_call(
        _tc_readout_body,
        out_shape=jax.ShapeDtypeStruct((B, D), _f32),
    )(p, g, dinv2, b2d, res, batch2d, wo_pad, bo2d)


def kernel(x, edge_index, batch, edge_attr, y, W1, b1, W2, b2, W3, b3,
           Wo, bo):
    E = edge_index.shape[1]
    pad_ids = N + jnp.arange(E2 - E, dtype=_i32) % (NPAD - N)
    r3 = jnp.concatenate([edge_index[0], pad_ids]).reshape(NW, NBATCH // 2,
                                                           2 * K)
    c3 = jnp.concatenate([edge_index[1], pad_ids]).reshape(NW, NBATCH // 2,
                                                           2 * K)
    w3 = jnp.pad(lax.bitcast_convert_type(edge_attr, _i32),
                 (0, E2 - E)).reshape(NW, NBATCH // 2, 2 * K)
    rc3 = (r3 << 16) | c3
    edata = jnp.stack([rc3, w3], axis=2).reshape(NW, TWORDS)
    edata = jnp.pad(edata, ((0, 0), (0, TPAD - TWORDS)))
    x_pad = jnp.pad(x, ((0, NPAD - N), (0, 0)))
    batch2d = jnp.pad(batch, (0, NPAD - N),
                      constant_values=jnp.int32(B + 1)).reshape(1, NPAD)
    wo_pad = jnp.pad(Wo, ((0, 0), (0, D - C)))
    bo2d = jnp.pad(bo, (0, D - C)).reshape(1, D)

    deg_parts = _sc_deg(edata)
    dinv2 = _tc_dinv(deg_parts).reshape(NPAD, 1)

    g1 = _tc_mm(x_pad, W1, dinv2)
    p1 = _sc_prop(g1, edata)
    t2, g2 = _tc_combine_mm(p1, g1, dinv2, b1.reshape(1, D), W2, None)
    p2 = _sc_prop(g2, edata)
    t3, g3 = _tc_combine_mm(p2, g2, dinv2, b2.reshape(1, D), W3, t2)
    p3 = _sc_prop(g3, edata)
    out = _tc_readout(p3, g3, dinv2, b3.reshape(1, D), t3,
                      batch2d, wo_pad, bo2d)
    return out[:, :C]


# speedup vs baseline: 1.1559x; 1.1559x over previous
import functools

import jax
import jax.numpy as jnp
from jax import lax
from jax.experimental import pallas as pl
from jax.experimental.pallas import tpu as pltpu
from jax.experimental.pallas import tpu_sc as plsc

N = 10000
NPAD = 10240
D = 128
B = 64
C = 10

NC = 2
NS = 16
NW = NC * NS

K = 32
EPT = 10240
NBATCH = EPT // K
E2 = NW * EPT
EROW = 2 * K
TWORDS = NBATCH * EROW
TPAD = 20736
ROWS_PT = NPAD // NS

_f32 = jnp.float32
_i32 = jnp.int32

_MESH = plsc.VectorSubcoreMesh(
    core_axis_name="c", subcore_axis_name="s", num_cores=NC, num_subcores=NS)


@functools.partial(
    pl.kernel,
    out_type=jax.ShapeDtypeStruct((NW, NPAD), _f32),
    mesh=_MESH,
    compiler_params=pltpu.CompilerParams(needs_layout_passes=False),
    scratch_types=[
        pltpu.VMEM((TPAD,), _i32),
        pltpu.VMEM((NPAD,), _f32),
    ],
)
def _sc_deg(edata_hbm, out_hbm, ed_t, deg_t):
    c = lax.axis_index("c")
    s = lax.axis_index("s")
    w = c * NS + s
    pltpu.sync_copy(edata_hbm.at[w], ed_t)

    zero16 = jnp.zeros((16,), _f32)

    def _zero(i, _):
        deg_t[pl.ds(pl.multiple_of(i * 16, 16), 16)] = zero16
        return _

    lax.fori_loop(0, NPAD // 16, _zero, None)

    def _acc(m, _):
        off = pl.multiple_of(m * EROW, 64)
        for q in range(K // 16):
            col_v = ed_t[pl.ds(off + q * 16, 16)] & 0xFFFF
            ew_v = plsc.bitcast(ed_t[pl.ds(off + K + q * 16, 16)], _f32)
            plsc.addupdate_scatter(deg_t, [col_v], ew_v)
        return _

    lax.fori_loop(0, NBATCH, _acc, None)
    pltpu.sync_copy(deg_t, out_hbm.at[w])


@functools.partial(
    pl.kernel,
    out_type=jax.ShapeDtypeStruct((NC, NPAD, D), _f32),
    mesh=_MESH,
    compiler_params=pltpu.CompilerParams(needs_layout_passes=False),
    scratch_types=[
        pltpu.VMEM((TPAD,), _i32),
        pltpu.VMEM((K, D), _f32),
        pltpu.VMEM((K, D), _f32),
        pltpu.VMEM((K, D), _f32),
        pltpu.VMEM((K, D), _f32),
        pltpu.VMEM((K, D), _f32),
        pltpu.VMEM((K, D), _f32),
        pltpu.VMEM((K,), _i32),
        pltpu.VMEM((K,), _i32),
        pltpu.VMEM((K,), _i32),
        pltpu.VMEM((K,), _i32),
        pltpu.VMEM((K,), _i32),
        pltpu.VMEM((K,), _i32),
        pltpu.VMEM_SHARED((NPAD, D), _f32),
        pltpu.SemaphoreType.DMA,
        pltpu.SemaphoreType.DMA,
        pltpu.SemaphoreType.DMA,
        pltpu.SemaphoreType.DMA,
        pltpu.SemaphoreType.DMA,
        pltpu.SemaphoreType.DMA,
        pltpu.SemaphoreType.DMA,
    ],
)
def _sc_prop(g_hbm, edata_hbm, out_hbm,
             ed_t, gb0, gb1, gb2, gb3, sb0, sb1,
             rb0, rb1, rb2, rb3, cb0, cb1, acc,
             gs0, gs1, gs2, gs3, ss0, ss1, es):
    c = lax.axis_index("c")
    s = lax.axis_index("s")
    w = c * NS + s

    gbufs = (gb0, gb1, gb2, gb3)
    rbufs = (rb0, rb1, rb2, rb3)
    gsems = (gs0, gs1, gs2, gs3)
    sbufs = (sb0, sb1)
    cbufs = (cb0, cb1)
    ssems = (ss0, ss1)

    pltpu.async_copy(edata_hbm.at[w], ed_t, es)

    zero16 = jnp.zeros((16,), _f32)
    for j in range(K):
        for t in range(D // 16):
            sl = pl.ds(t * 16, 16)
            sb0[j, sl] = zero16
            sb1[j, sl] = zero16
    zbase = s * ROWS_PT
    for t in range(ROWS_PT // K):
        pltpu.sync_copy(sbufs[t % 2], acc.at[pl.ds(zbase + t * K, K)])
    plsc.subcore_barrier()

    pltpu.make_async_copy(edata_hbm.at[w], ed_t, es).wait()

    def _start_gather(u, m):
        off = pl.multiple_of(m * EROW, 64)
        for q in range(K // 16):
            rbufs[u][pl.ds(q * 16, 16)] = (
                ed_t[pl.ds(off + q * 16, 16)] >> 16)
        pltpu.async_copy(g_hbm.at[rbufs[u]], gbufs[u], gsems[u])

    _start_gather(0, 0)
    _start_gather(1, 1)
    _start_gather(2, 2)

    def _body(i, _):
        for u in range(4):
            m = i * 4 + u
            p = u % 2
            off = pl.multiple_of(m * EROW, 64)

            _start_gather((u + 3) % 4, m + 3)

            pltpu.make_async_copy(g_hbm.at[rbufs[u]], gbufs[u], gsems[u]).wait()

            def _wait_scatter(p=p):
                pltpu.make_async_copy(
                    sbufs[p], acc.at[cbufs[p]], ssems[p]).wait()
            if u >= 2:
                _wait_scatter()
            else:
                pl.when(i >= 1)(_wait_scatter)

            for j in range(K):
                if j % 16 == 0:
                    ew_v = plsc.bitcast(
                        ed_t[pl.ds(off + K + j, 16)], _f32)
                wj = jnp.full((16,), ew_v[j % 16], _f32)
                for t in range(D // 16):
                    sl = pl.ds(t * 16, 16)
                    sbufs[p][j, sl] = gbufs[u][j, sl] * wj

            for q in range(K // 16):
                cbufs[p][pl.ds(q * 16, 16)] = (
                    ed_t[pl.ds(off + q * 16, 16)] & 0xFFFF)

            pltpu.async_copy(sbufs[p], acc.at[cbufs[p]], ssems[p], add=True)
        return _

    lax.fori_loop(0, NBATCH // 4, _body, None)

    for p in range(2):
        pltpu.make_async_copy(sbufs[p], acc.at[cbufs[p]], ssems[p]).wait()
    for u in range(3):
        pltpu.make_async_copy(g_hbm.at[rbufs[u]], gbufs[u], gsems[u]).wait()

    plsc.subcore_barrier()
    dbase = s * ROWS_PT
    pltpu.sync_copy(acc.at[pl.ds(dbase, ROWS_PT)],
                    out_hbm.at[c, pl.ds(dbase, ROWS_PT)])


def _tc_dinv_body(parts_ref, out_ref):
    deg = jnp.sum(parts_ref[...], axis=0) + 1.0
    out_ref[...] = lax.rsqrt(deg)


def _tc_dinv(parts):
    return pl.pallas_call(
        _tc_dinv_body,
        out_shape=jax.ShapeDtypeStruct((NPAD,), _f32),
    )(parts)


_BM = 1024
_GRID = NPAD // _BM


def _tc_mm_body(x_ref, w_ref, dinv_ref, g_ref):
    g_ref[...] = dinv_ref[...] * jnp.dot(
        x_ref[...], w_ref[...], preferred_element_type=_f32)


def _tc_mm(x, w, dinv2):
    return pl.pallas_call(
        _tc_mm_body,
        grid=(_GRID,),
        in_specs=[
            pl.BlockSpec((_BM, D), lambda i: (i, 0)),
            pl.BlockSpec((D, D), lambda i: (0, 0)),
            pl.BlockSpec((_BM, 1), lambda i: (i, 0)),
        ],
        out_specs=pl.BlockSpec((_BM, D), lambda i: (i, 0)),
        out_shape=jax.ShapeDtypeStruct((NPAD, D), _f32),
    )(x, w, dinv2)


def _tc_combine_mm_body(with_res, p_ref, g_ref, dinv_ref, b_ref, w_ref,
                        res_ref, t_ref, gn_ref):
    ps = p_ref[0] + p_ref[1]
    t = dinv_ref[...] * (ps + g_ref[...]) + b_ref[...]
    if with_res:
        t = t + res_ref[...]
    t = jnp.maximum(t, 0.0)
    t_ref[...] = t
    gn_ref[...] = dinv_ref[...] * jnp.dot(
        t, w_ref[...], preferred_element_type=_f32)


def _tc_combine_mm(p, g, dinv2, b2d, w, res):
    with_res = res is not None
    specs = [
        pl.BlockSpec((NC, _BM, D), lambda i: (0, i, 0)),
        pl.BlockSpec((_BM, D), lambda i: (i, 0)),
        pl.BlockSpec((_BM, 1), lambda i: (i, 0)),
        pl.BlockSpec((1, D), lambda i: (0, 0)),
        pl.BlockSpec((D, D), lambda i: (0, 0)),
    ]
    args = [p, g, dinv2, b2d, w]
    if with_res:
        specs.append(pl.BlockSpec((_BM, D), lambda i: (i, 0)))
        args.append(res)
        body = lambda p_, g_, d_, b_, w_, r_, t_, gn_: _tc_combine_mm_body(
            True, p_, g_, d_, b_, w_, r_, t_, gn_)
    else:
        body = lambda p_, g_, d_, b_, w_, t_, gn_: _tc_combine_mm_body(
            False, p_, g_, d_, b_, w_, None, t_, gn_)
    return pl.pallas_call(
        body,
        grid=(_GRID,),
        in_specs=specs,
        out_specs=[
            pl.BlockSpec((_BM, D), lambda i: (i, 0)),
            pl.BlockSpec((_BM, D), lambda i: (i, 0)),
        ],
        out_shape=[
            jax.ShapeDtypeStruct((NPAD, D), _f32),
            jax.ShapeDtypeStruct((NPAD, D), _f32),
        ],
    )(*args)


def _tc_combine_body(p_ref, g_ref, dinv_ref, b_ref, res_ref, h_ref):
    ps = p_ref[0] + p_ref[1]
    t = res_ref[...] + dinv_ref[...] * (ps + g_ref[...]) + b_ref[...]
    h_ref[...] = jnp.maximum(t, 0.0)


def _tc_combine(p, g, dinv2, b2d, res):
    return pl.pallas_call(
        _tc_combine_body,
        grid=(_GRID,),
        in_specs=[
            pl.BlockSpec((NC, _BM, D), lambda i: (0, i, 0)),
            pl.BlockSpec((_BM, D), lambda i: (i, 0)),
            pl.BlockSpec((_BM, 1), lambda i: (i, 0)),
            pl.BlockSpec((1, D), lambda i: (0, 0)),
            pl.BlockSpec((_BM, D), lambda i: (i, 0)),
        ],
        out_specs=pl.BlockSpec((_BM, D), lambda i: (i, 0)),
        out_shape=jax.ShapeDtypeStruct((NPAD, D), _f32),
    )(p, g, dinv2, b2d, res)


def _tc_readout_body(p_ref, g_ref, dinv_ref, b_ref, res_ref,
                     batch_ref, wo_ref, bo_ref, out_ref):
    t = (res_ref[...] + dinv_ref[...] * (p_ref[0] + p_ref[1] + g_ref[...])
         + b_ref[...])
    h = jnp.maximum(t, 0.0)
    bat = batch_ref[...]
    karr = lax.broadcasted_iota(_i32, (B, 1), 0)
    cnt = jnp.sum((bat <= karr).astype(_i32), axis=1, keepdims=True)
    pos = lax.broadcasted_iota(_i32, (B, NPAD), 1)
    sel = (pos == (cnt - 1)).astype(_f32)
    virt = jnp.dot(sel, h, preferred_element_type=_f32)
    out_ref[...] = jnp.dot(
        virt, wo_ref[...], preferred_element_type=_f32) + bo_ref[...]


def _tc_readout(p, g, dinv2, b2d, res, batch2d, wo_pad, bo2d):
    return pl.pallas_call(
        _tc_readout_body,
        out_shape=jax.ShapeDtypeStruct((B, D), _f32),
    )(p, g, dinv2, b2d, res, batch2d, wo_pad, bo2d)


def kernel(x, edge_index, batch, edge_attr, y, W1, b1, W2, b2, W3, b3,
           Wo, bo):
    E = edge_index.shape[1]
    pad_ids = N + jnp.arange(E2 - E, dtype=_i32) % (NPAD - N)
    r3 = jnp.concatenate([edge_index[0], pad_ids]).reshape(NW, NBATCH, K)
    c3 = jnp.concatenate([edge_index[1], pad_ids]).reshape(NW, NBATCH, K)
    w3 = jnp.pad(lax.bitcast_convert_type(edge_attr, _i32),
                 (0, E2 - E)).reshape(NW, NBATCH, K)
    rc3 = (r3 << 16) | c3
    edata = jnp.stack([rc3, w3], axis=2).reshape(NW, TWORDS)
    edata = jnp.pad(edata, ((0, 0), (0, TPAD - TWORDS)))
    x_pad = jnp.pad(x, ((0, NPAD - N), (0, 0)))
    batch2d = jnp.pad(batch, (0, NPAD - N),
                      constant_values=jnp.int32(B + 1)).reshape(1, NPAD)
    wo_pad = jnp.pad(Wo, ((0, 0), (0, D - C)))
    bo2d = jnp.pad(bo, (0, D - C)).reshape(1, D)

    deg_parts = _sc_deg(edata)
    dinv2 = _tc_dinv(deg_parts).reshape(NPAD, 1)

    g1 = _tc_mm(x_pad, W1, dinv2)
    p1 = _sc_prop(g1, edata)
    t2, g2 = _tc_combine_mm(p1, g1, dinv2, b1.reshape(1, D), W2, None)
    p2 = _sc_prop(g2, edata)
    t3, g3 = _tc_combine_mm(p2, g2, dinv2, b2.reshape(1, D), W3, t2)
    p3 = _sc_prop(g3, edata)
    out = _tc_readout(p3, g3, dinv2, b3.reshape(1, D), t3,
                      batch2d, wo_pad, bo2d)
    return out[:, :C]


# speedup vs baseline: 1.3491x; 1.1671x over previous
import functools

import jax
import jax.numpy as jnp
from jax import lax
from jax.experimental import pallas as pl
from jax.experimental.pallas import tpu as pltpu
from jax.experimental.pallas import tpu_sc as plsc

N = 10000
NPAD = 10240
D = 128
B = 64
C = 10

NC = 2
NS = 16
NW = NC * NS

K = 32
EPT = 10240
NBATCH = EPT // K
E2 = NW * EPT
EROW = 2 * K
TWORDS = NBATCH * EROW
TPAD = 20736
ROWS_PT = NPAD // NS

_f32 = jnp.float32
_i32 = jnp.int32

_MESH = plsc.VectorSubcoreMesh(
    core_axis_name="c", subcore_axis_name="s", num_cores=NC, num_subcores=NS)


@functools.partial(
    pl.kernel,
    out_type=jax.ShapeDtypeStruct((NW, NPAD), _f32),
    mesh=_MESH,
    compiler_params=pltpu.CompilerParams(needs_layout_passes=False),
    scratch_types=[
        pltpu.VMEM((TPAD,), _i32),
        pltpu.VMEM((NPAD,), _f32),
    ],
)
def _sc_deg(edata_hbm, out_hbm, ed_t, deg_t):
    c = lax.axis_index("c")
    s = lax.axis_index("s")
    w = c * NS + s
    pltpu.sync_copy(edata_hbm.at[w], ed_t)

    zero16 = jnp.zeros((16,), _f32)

    def _zero(i, _):
        deg_t[pl.ds(pl.multiple_of(i * 16, 16), 16)] = zero16
        return _

    lax.fori_loop(0, NPAD // 16, _zero, None)

    def _acc(m, _):
        off = pl.multiple_of(m * EROW, 64)
        for q in range(K // 16):
            col_v = ed_t[pl.ds(off + q * 16, 16)] & 0xFFFF
            ew_v = plsc.bitcast(ed_t[pl.ds(off + K + q * 16, 16)], _f32)
            plsc.addupdate_scatter(deg_t, [col_v], ew_v)
        return _

    lax.fori_loop(0, NBATCH, _acc, None)
    pltpu.sync_copy(deg_t, out_hbm.at[w])


@functools.partial(
    pl.kernel,
    out_type=jax.ShapeDtypeStruct((NC, NPAD, D), _f32),
    mesh=_MESH,
    compiler_params=pltpu.CompilerParams(needs_layout_passes=False,
                                         use_tc_tiling_on_sc=False),
    scratch_types=[
        pltpu.VMEM((TPAD,), _i32),
        pltpu.VMEM((K, D // 2), _i32),
        pltpu.VMEM((K, D // 2), _i32),
        pltpu.VMEM((K, D // 2), _i32),
        pltpu.VMEM((K, D // 2), _i32),
        pltpu.VMEM((K, D), _f32),
        pltpu.VMEM((K, D), _f32),
        pltpu.VMEM((K,), _i32),
        pltpu.VMEM((K,), _i32),
        pltpu.VMEM((K,), _i32),
        pltpu.VMEM((K,), _i32),
        pltpu.VMEM((K,), _i32),
        pltpu.VMEM((K,), _i32),
        pltpu.VMEM_SHARED((NPAD, D), _f32),
        pltpu.SemaphoreType.DMA,
        pltpu.SemaphoreType.DMA,
        pltpu.SemaphoreType.DMA,
        pltpu.SemaphoreType.DMA,
        pltpu.SemaphoreType.DMA,
        pltpu.SemaphoreType.DMA,
        pltpu.SemaphoreType.DMA,
    ],
)
def _sc_prop(gi_hbm, edata_hbm, out_hbm,
             ed_t, gb0, gb1, gb2, gb3, sb0, sb1,
             rb0, rb1, rb2, rb3, cb0, cb1, acc,
             gs0, gs1, gs2, gs3, ss0, ss1, es):
    c = lax.axis_index("c")
    s = lax.axis_index("s")
    w = c * NS + s

    gbufs = (gb0, gb1, gb2, gb3)
    rbufs = (rb0, rb1, rb2, rb3)
    gsems = (gs0, gs1, gs2, gs3)
    sbufs = (sb0, sb1)
    cbufs = (cb0, cb1)
    ssems = (ss0, ss1)

    pltpu.async_copy(edata_hbm.at[w], ed_t, es)

    zero16 = jnp.zeros((16,), _f32)
    for j in range(K):
        for t in range(D // 16):
            sl = pl.ds(t * 16, 16)
            sb0[j, sl] = zero16
            sb1[j, sl] = zero16
    zbase = s * ROWS_PT
    for t in range(ROWS_PT // K):
        pltpu.sync_copy(sbufs[t % 2], acc.at[pl.ds(zbase + t * K, K)])
    plsc.subcore_barrier()

    pltpu.make_async_copy(edata_hbm.at[w], ed_t, es).wait()

    def _start_gather(u, m):
        off = pl.multiple_of(m * EROW, 64)
        for q in range(K // 16):
            rbufs[u][pl.ds(q * 16, 16)] = (
                ed_t[pl.ds(off + q * 16, 16)] >> 16)
        pltpu.async_copy(gi_hbm.at[rbufs[u]], gbufs[u], gsems[u])

    _start_gather(0, 0)
    _start_gather(1, 1)
    _start_gather(2, 2)

    def _body(i, _):
        for u in range(4):
            m = i * 4 + u
            p = u % 2
            off = pl.multiple_of(m * EROW, 64)

            _start_gather((u + 3) % 4, m + 3)

            pltpu.make_async_copy(
                gi_hbm.at[rbufs[u]], gbufs[u], gsems[u]).wait()

            def _wait_scatter(p=p):
                pltpu.make_async_copy(
                    sbufs[p], acc.at[cbufs[p]], ssems[p]).wait()
            if u >= 2:
                _wait_scatter()
            else:
                pl.when(i >= 1)(_wait_scatter)

            for j in range(K):
                if j % 16 == 0:
                    ew_v = plsc.bitcast(
                        ed_t[pl.ds(off + K + j, 16)], _f32)
                wj = jnp.full((16,), ew_v[j % 16], _f32)
                for t in range(D // 32):
                    v = gbufs[u][j, pl.ds(t * 16, 16)]
                    lo = plsc.bitcast(v << 16, _f32)
                    hi = plsc.bitcast(v & jnp.int32(-65536), _f32)
                    sbufs[p][j, pl.ds(t * 32, 16)] = lo * wj
                    sbufs[p][j, pl.ds(t * 32 + 16, 16)] = hi * wj

            for q in range(K // 16):
                cbufs[p][pl.ds(q * 16, 16)] = (
                    ed_t[pl.ds(off + q * 16, 16)] & 0xFFFF)

            pltpu.async_copy(sbufs[p], acc.at[cbufs[p]], ssems[p], add=True)
        return _

    lax.fori_loop(0, NBATCH // 4, _body, None)

    for p in range(2):
        pltpu.make_async_copy(sbufs[p], acc.at[cbufs[p]], ssems[p]).wait()
    for u in range(3):
        pltpu.make_async_copy(gi_hbm.at[rbufs[u]], gbufs[u], gsems[u]).wait()

    plsc.subcore_barrier()
    dbase = s * ROWS_PT
    pltpu.sync_copy(acc.at[pl.ds(dbase, ROWS_PT)],
                    out_hbm.at[c, pl.ds(dbase, ROWS_PT)])


def _tc_dinv_body(parts_ref, out_ref):
    deg = jnp.sum(parts_ref[...], axis=0) + 1.0
    out_ref[...] = lax.rsqrt(deg)


def _tc_dinv(parts):
    return pl.pallas_call(
        _tc_dinv_body,
        out_shape=jax.ShapeDtypeStruct((NPAD,), _f32),
    )(parts)


_BM = 1024
_GRID = NPAD // _BM


def _tc_mm_body(x_ref, w_ref, wp_ref, dinv_ref, g_ref, gb_ref):
    x = x_ref[...]
    dinv = dinv_ref[...]
    g_ref[...] = dinv * jnp.dot(x, w_ref[...], preferred_element_type=_f32)
    gb_ref[...] = (dinv * jnp.dot(
        x, wp_ref[...], preferred_element_type=_f32)).astype(jnp.bfloat16)


def _tc_mm(x, w, wp, dinv2):
    return pl.pallas_call(
        _tc_mm_body,
        grid=(_GRID,),
        in_specs=[
            pl.BlockSpec((_BM, D), lambda i: (i, 0)),
            pl.BlockSpec((D, D), lambda i: (0, 0)),
            pl.BlockSpec((D, D), lambda i: (0, 0)),
            pl.BlockSpec((_BM, 1), lambda i: (i, 0)),
        ],
        out_specs=[
            pl.BlockSpec((_BM, D), lambda i: (i, 0)),
            pl.BlockSpec((_BM, D), lambda i: (i, 0)),
        ],
        out_shape=[
            jax.ShapeDtypeStruct((NPAD, D), _f32),
            jax.ShapeDtypeStruct((NPAD, D), jnp.bfloat16),
        ],
    )(x, w, wp, dinv2)


def _tc_combine_mm_body(with_res, p_ref, g_ref, dinv_ref, b_ref, w_ref,
                        wp_ref, res_ref, t_ref, gn_ref, gnb_ref):
    ps = p_ref[0] + p_ref[1]
    dinv = dinv_ref[...]
    t = dinv * (ps + g_ref[...]) + b_ref[...]
    if with_res:
        t = t + res_ref[...]
    t = jnp.maximum(t, 0.0)
    t_ref[...] = t
    gn_ref[...] = dinv * jnp.dot(t, w_ref[...], preferred_element_type=_f32)
    gnb_ref[...] = (dinv * jnp.dot(
        t, wp_ref[...], preferred_element_type=_f32)).astype(jnp.bfloat16)


def _tc_combine_mm(p, g, dinv2, b2d, w, wp, res):
    with_res = res is not None
    specs = [
        pl.BlockSpec((NC, _BM, D), lambda i: (0, i, 0)),
        pl.BlockSpec((_BM, D), lambda i: (i, 0)),
        pl.BlockSpec((_BM, 1), lambda i: (i, 0)),
        pl.BlockSpec((1, D), lambda i: (0, 0)),
        pl.BlockSpec((D, D), lambda i: (0, 0)),
        pl.BlockSpec((D, D), lambda i: (0, 0)),
    ]
    args = [p, g, dinv2, b2d, w, wp]
    if with_res:
        specs.append(pl.BlockSpec((_BM, D), lambda i: (i, 0)))
        args.append(res)
        body = lambda p_, g_, d_, b_, w_, wp_, r_, t_, gn_, gnb_: (
            _tc_combine_mm_body(True, p_, g_, d_, b_, w_, wp_, r_,
                                t_, gn_, gnb_))
    else:
        body = lambda p_, g_, d_, b_, w_, wp_, t_, gn_, gnb_: (
            _tc_combine_mm_body(False, p_, g_, d_, b_, w_, wp_, None,
                                t_, gn_, gnb_))
    return pl.pallas_call(
        body,
        grid=(_GRID,),
        in_specs=specs,
        out_specs=[
            pl.BlockSpec((_BM, D), lambda i: (i, 0)),
            pl.BlockSpec((_BM, D), lambda i: (i, 0)),
            pl.BlockSpec((_BM, D), lambda i: (i, 0)),
        ],
        out_shape=[
            jax.ShapeDtypeStruct((NPAD, D), _f32),
            jax.ShapeDtypeStruct((NPAD, D), _f32),
            jax.ShapeDtypeStruct((NPAD, D), jnp.bfloat16),
        ],
    )(*args)


def _tc_combine_body(p_ref, g_ref, dinv_ref, b_ref, res_ref, h_ref):
    ps = p_ref[0] + p_ref[1]
    t = res_ref[...] + dinv_ref[...] * (ps + g_ref[...]) + b_ref[...]
    h_ref[...] = jnp.maximum(t, 0.0)


def _tc_combine(p, g, dinv2, b2d, res):
    return pl.pallas_call(
        _tc_combine_body,
        grid=(_GRID,),
        in_specs=[
            pl.BlockSpec((NC, _BM, D), lambda i: (0, i, 0)),
            pl.BlockSpec((_BM, D), lambda i: (i, 0)),
            pl.BlockSpec((_BM, 1), lambda i: (i, 0)),
            pl.BlockSpec((1, D), lambda i: (0, 0)),
            pl.BlockSpec((_BM, D), lambda i: (i, 0)),
        ],
        out_specs=pl.BlockSpec((_BM, D), lambda i: (i, 0)),
        out_shape=jax.ShapeDtypeStruct((NPAD, D), _f32),
    )(p, g, dinv2, b2d, res)


def _tc_readout_body(p_ref, g_ref, dinv_ref, b_ref, res_ref,
                     batch_ref, wo_ref, bo_ref, out_ref):
    t = (res_ref[...] + dinv_ref[...] * (p_ref[0] + p_ref[1] + g_ref[...])
         + b_ref[...])
    h = jnp.maximum(t, 0.0)
    bat = batch_ref[...]
    karr = lax.broadcasted_iota(_i32, (B, 1), 0)
    cnt = jnp.sum((bat <= karr).astype(_i32), axis=1, keepdims=True)
    pos = lax.broadcasted_iota(_i32, (B, NPAD), 1)
    sel = (pos == (cnt - 1)).astype(_f32)
    virt = jnp.dot(sel, h, preferred_element_type=_f32)
    out_ref[...] = jnp.dot(
        virt, wo_ref[...], preferred_element_type=_f32) + bo_ref[...]


def _tc_readout(p, g, dinv2, b2d, res, batch2d, wo_pad, bo2d):
    return pl.pallas_call(
        _tc_readout_body,
        out_shape=jax.ShapeDtypeStruct((B, D), _f32),
    )(p, g, dinv2, b2d, res, batch2d, wo_pad, bo2d)


def kernel(x, edge_index, batch, edge_attr, y, W1, b1, W2, b2, W3, b3,
           Wo, bo):
    E = edge_index.shape[1]
    pad_ids = N + jnp.arange(E2 - E, dtype=_i32) % (NPAD - N)
    r3 = jnp.concatenate([edge_index[0], pad_ids]).reshape(NW, NBATCH, K)
    c3 = jnp.concatenate([edge_index[1], pad_ids]).reshape(NW, NBATCH, K)
    w3 = jnp.pad(lax.bitcast_convert_type(edge_attr, _i32),
                 (0, E2 - E)).reshape(NW, NBATCH, K)
    rc3 = (r3 << 16) | c3
    edata = jnp.stack([rc3, w3], axis=2).reshape(NW, TWORDS)
    edata = jnp.pad(edata, ((0, 0), (0, TPAD - TWORDS)))
    x_pad = jnp.pad(x, ((0, NPAD - N), (0, 0)))
    batch2d = jnp.pad(batch, (0, NPAD - N),
                      constant_values=jnp.int32(B + 1)).reshape(1, NPAD)
    wo_pad = jnp.pad(Wo, ((0, 0), (0, D - C)))
    bo2d = jnp.pad(bo, (0, D - C)).reshape(1, D)

    pidx = []
    for qq in range(D // 32):
        for kk in range(16):
            pidx += [32 * qq + kk, 32 * qq + 16 + kk]
    pidx = jnp.array(pidx, dtype=_i32)

    def _as_i32(gb):
        return lax.bitcast_convert_type(gb.reshape(NPAD, D // 2, 2), _i32)

    deg_parts = _sc_deg(edata)
    dinv2 = _tc_dinv(deg_parts).reshape(NPAD, 1)

    g1, g1b = _tc_mm(x_pad, W1, W1[:, pidx], dinv2)
    p1 = _sc_prop(_as_i32(g1b), edata)
    t2, g2, g2b = _tc_combine_mm(p1, g1, dinv2, b1.reshape(1, D),
                                 W2, W2[:, pidx], None)
    p2 = _sc_prop(_as_i32(g2b), edata)
    t3, g3, g3b = _tc_combine_mm(p2, g2, dinv2, b2.reshape(1, D),
                                 W3, W3[:, pidx], t2)
    p3 = _sc_prop(_as_i32(g3b), edata)
    out = _tc_readout(p3, g3, dinv2, b3.reshape(1, D), t3,
                      batch2d, wo_pad, bo2d)
    return out[:, :C]
